# baseline (speedup 1.0000x reference)
import functools

import jax
import jax.numpy as jnp
from jax import lax
from jax.experimental import pallas as pl
from jax.experimental.pallas import tpu as pltpu
from jax.experimental.pallas import tpu_sc as plsc

B = 4096
L = 200
EMB = 128
VOCAB = 100000

NW = 32
N = B * L
PER_W = N // NW
CHUNK = 128
SLAB = 4096
NPASS = 25
CAP = 1280
IDXCH = 3200
NIDXCH = PER_W // IDXCH

_mesh = plsc.VectorSubcoreMesh(core_axis_name="c", subcore_axis_name="s")


@functools.partial(
    pl.kernel,
    out_type=jax.ShapeDtypeStruct((N, EMB), jnp.float32),
    mesh=_mesh,
    compiler_params=pltpu.CompilerParams(needs_layout_passes=False),
    scratch_types=[
        pltpu.VMEM((NPASS * CAP,), jnp.int32),
        pltpu.VMEM((IDXCH,), jnp.int32),
        pltpu.VMEM((IDXCH,), jnp.int32),
        pltpu.VMEM((CHUNK,), jnp.int32),
        pltpu.VMEM((CHUNK,), jnp.int32),
        pltpu.VMEM((CHUNK,), jnp.int32),
        pltpu.VMEM((CHUNK,), jnp.int32),
        pltpu.VMEM((2, CHUNK), jnp.int32),
        pltpu.VMEM((32,), jnp.int32),
        pltpu.VMEM((CHUNK, EMB), jnp.float32),
        pltpu.VMEM((CHUNK, EMB), jnp.float32),
        pltpu.VMEM_SHARED((SLAB, EMB), jnp.float32),
        pltpu.SemaphoreType.DMA,
        pltpu.SemaphoreType.DMA,
        pltpu.SemaphoreType.DMA,
        pltpu.SemaphoreType.DMA,
    ],
)
def _gather_kernel(idx_hbm, table_hbm, out_hbm, ppack, ibuf0, ibuf1,
                   glist0, glist1, slist0, slist1, padpos, counts,
                   rows0, rows1, slab_sh, gsem0, gsem1, wsem0, wsem1):
    wid = lax.axis_index("s") * 2 + lax.axis_index("c")
    sid = lax.axis_index("s")
    base = wid * PER_W
    iota = lax.iota(jnp.int32, 16)
    rows = (rows0, rows1)
    ibufs = (ibuf0, ibuf1)
    glists = (glist0, glist1)
    slists = (slist0, slist1)
    wsems = (wsem0, wsem1)

    def vext(vec, lane):
        return lax.reduce_max(
            jnp.where(iota == lane, vec, jnp.int32(-2147483648)), (0,))

    for k in range(2):
        for j in range(8):
            plsc.store_scatter(padpos, [iota * 0 + k, j * 16 + iota],
                               iota * 0 + base)
    for k in range(2):
        pltpu.async_copy(rows[k], out_hbm.at[padpos.at[k]], wsems[k])

    def wait_write(k):
        pltpu.make_async_copy(rows[k], out_hbm.at[padpos.at[k]], wsems[k]).wait()

    counts[pl.ds(0, 16)] = iota * CAP
    counts[pl.ds(16, 16)] = (iota + 16) * CAP

    for c in range(NIDXCH):
        buf = ibufs[c % 2]
        pltpu.sync_copy(
            idx_hbm.at[pl.ds(wid * PER_W + c * IDXCH, IDXCH)], buf)

        @pl.loop(0, IDXCH // 16)
        def _part(g):
            v = buf[pl.ds(g * 16, 16)]
            b = lax.shift_right_logical(v, 12)
            r = v & (SLAB - 1)
            lpos = c * IDXCH + g * 16 + iota
            occ, lastm = plsc.scan_count(b)
            cur = plsc.load_gather(counts, [b])
            slot = cur + occ - 1
            okm = slot < (b + 1) * CAP
            plsc.store_scatter(ppack, [slot],
                               lax.shift_left(r, 15) | lpos, mask=okm)
            plsc.store_scatter(counts, [b], cur + occ, mask=lastm)

    c_lo = counts[pl.ds(0, 16)]
    c_hi = counts[pl.ds(16, 16)]
    size_lo = c_lo - iota * CAP
    size_hi = c_hi - (iota + 16) * CAP
    n_over = (plsc.all_reduce_population_count(size_lo > CAP)
              + plsc.all_reduce_population_count(size_hi > CAP))
    overflow = lax.reduce_max(n_over, (0,)) > 0

    @pl.when(jnp.logical_not(overflow))
    def _main():
        for p in range(NPASS):
            cval = vext(c_lo, p) if p < 16 else vext(c_hi, p - 16)
            cnt = cval - p * CAP
            nch = (cnt + 127) // CHUNK
            start = p * CAP + cnt
            end = p * CAP + nch * CHUNK
            pad_val = vext(ppack[pl.ds(p * CAP, 16)], 0)
            for j in range(8):
                ids = start + j * 16 + iota
                m = ids < end
                plsc.store_scatter(ppack, [ids], iota * 0 + pad_val, mask=m)

        for p in range(NPASS):
            if p < NPASS - 1:
                share = SLAB // 16
                pltpu.sync_copy(
                    table_hbm.at[pl.ds(p * SLAB + sid * share, share)],
                    slab_sh.at[pl.ds(sid * share, share)])
            else:
                @pl.when(sid < 15)
                def _most():
                    pltpu.sync_copy(
                        table_hbm.at[pl.ds(p * SLAB + sid * 112, 112)],
                        slab_sh.at[pl.ds(sid * 112, 112)])

                @pl.when(sid == 15)
                def _last():
                    pltpu.sync_copy(
                        table_hbm.at[pl.ds(p * SLAB + 1680, 16)],
                        slab_sh.at[pl.ds(1680, 16)])
            plsc.subcore_barrier()

            cval = vext(c_lo, p) if p < 16 else vext(c_hi, p - 16)
            cnt = cval - p * CAP
            nch = (cnt + 127) // CHUNK

            def unpack(j, k):
                for t in range(CHUNK // 16):
                    pk = ppack[pl.ds(p * CAP + j * CHUNK + t * 16, 16)]
                    glists[k][pl.ds(t * 16, 16)] = lax.shift_right_logical(
                        pk, 15)
                    slists[k][pl.ds(t * 16, 16)] = (pk & 32767) + base

            @pl.loop(0, nch, step=2)
            def _chunks(j):
                wait_write(0)
                unpack(j, 0)
                d0 = pltpu.async_copy(slab_sh.at[glist0], rows0, gsem0)

                @pl.when(j + 1 < nch)
                def _odd_fetch():
                    wait_write(1)
                    unpack(j + 1, 1)
                    pltpu.async_copy(slab_sh.at[glist1], rows1, gsem1)

                d0.wait()
                pltpu.async_copy(rows0, out_hbm.at[slist0], wsem0)

                @pl.when(j + 1 < nch)
                def _odd_store():
                    pltpu.make_async_copy(
                        slab_sh.at[glist1], rows1, gsem1).wait()
                    pltpu.async_copy(rows1, out_hbm.at[slist1], wsem1)

            plsc.subcore_barrier()

        wait_write(0)
        wait_write(1)

    @pl.when(overflow)
    def _fallback():
        wait_write(0)
        wait_write(1)
        for c in range(NIDXCH):
            pltpu.sync_copy(
                idx_hbm.at[pl.ds(wid * PER_W + c * IDXCH, IDXCH)], ibuf0)

            @pl.loop(0, IDXCH // CHUNK)
            def _direct(g):
                pltpu.async_copy(
                    table_hbm.at[ibuf0.at[pl.ds(g * CHUNK, CHUNK)]], rows0,
                    gsem0)
                pltpu.make_async_copy(
                    table_hbm.at[pl.ds(0, CHUNK)], rows0, gsem0).wait()
                pltpu.sync_copy(
                    rows0,
                    out_hbm.at[pl.ds(base + c * IDXCH + g * CHUNK, CHUNK)])


def kernel(features, table):
    idx = features.reshape(N)
    out = _gather_kernel(idx, table)
    return out.reshape(B, L, EMB)


# speedup vs baseline: 1.3709x; 1.3709x over previous
import functools

import jax
import jax.numpy as jnp
from jax import lax
from jax.experimental import pallas as pl
from jax.experimental.pallas import tpu as pltpu
from jax.experimental.pallas import tpu_sc as plsc

B = 4096
L = 200
EMB = 128

NW = 32
N = B * L
PER_W = N // NW
CHUNK = 128
NCHUNK = PER_W // CHUNK
NBUF = 4

_mesh = plsc.VectorSubcoreMesh(core_axis_name="c", subcore_axis_name="s")


@functools.partial(
    pl.kernel,
    out_type=jax.ShapeDtypeStruct((N, EMB), jnp.float32),
    mesh=_mesh,
    scratch_types=[
        pltpu.VMEM((NCHUNK, CHUNK), jnp.int32),
        [pltpu.VMEM((CHUNK, EMB), jnp.float32)] * NBUF,
        [pltpu.SemaphoreType.DMA] * NBUF,
        [pltpu.SemaphoreType.DMA] * NBUF,
    ],
)
def _gather_kernel(idx_hbm, table_hbm, out_hbm, idx_v, rows, gsems, wsems):
    wid = lax.axis_index("s") * 2 + lax.axis_index("c")
    base = wid * PER_W

    pltpu.sync_copy(idx_hbm.at[pl.ds(wid * NCHUNK, NCHUNK)], idx_v)

    def issue_gather(g, k):
        pltpu.async_copy(table_hbm.at[idx_v.at[g]], rows[k], gsems[k])

    def wait_gather(k):
        pltpu.make_async_copy(table_hbm.at[pl.ds(0, CHUNK)], rows[k], gsems[k]).wait()

    def issue_write(g, k):
        pltpu.async_copy(rows[k], out_hbm.at[pl.ds(base + g * CHUNK, CHUNK)], wsems[k])

    def wait_write(k):
        pltpu.make_async_copy(rows[k], out_hbm.at[pl.ds(base, CHUNK)], wsems[k]).wait()

    for k in range(NBUF):
        issue_gather(k, k)

    @pl.loop(0, NCHUNK, step=NBUF)
    def _body(g):
        for k in range(NBUF):
            wait_gather(k)
            issue_write(g + k, k)
        for k in range(NBUF):
            @pl.when(g + NBUF + k < NCHUNK)
            def _():
                wait_write(k)
                issue_gather(g + NBUF + k, k)

    for k in range(NBUF):
        wait_write(k)


def kernel(features, table):
    idx = features.reshape(NW * NCHUNK, CHUNK)
    out = _gather_kernel(idx, table)
    return out.reshape(B, L, EMB)
